# Initial kernel scaffold; baseline (speedup 1.0000x reference)
import jax
import jax.numpy as jnp
from jax.experimental import pallas as pl


def kernel(x, edge_index, neg_edge_index, Wl1, Wr1, att1, b1, Wl2, Wr2, att2, b2, Wl3, Wr3, att3, b3, Wlin1, blin1, Wlin2, blin2, c1, c2):
    raise NotImplementedError("write your pallas kernel here")


# speedup vs baseline: 5.5465x; 5.5465x over previous
import functools

import jax
import jax.numpy as jnp
from jax import lax
from jax.experimental import pallas as pl
from jax.experimental.pallas import tpu as pltpu
from jax.experimental.pallas import tpu_sc as plsc

N = 10000
D = 128
E = 160000
NP = 10240
RPT = NP // 16
ZR = 64
B = 64
ETOT = E + N
ETP = 172032
EPT = ETP // 32
NCH = EPT // B
EPS = ETP // 16
NCHS = EPS // B
ELP = 321536
LPT = ELP // 32
LCH = LPT // B

_MESH = plsc.VectorSubcoreMesh(core_axis_name="c", subcore_axis_name="s")
_SC_PARAMS = pltpu.CompilerParams(needs_layout_passes=False)
_f32 = jnp.float32


def _zero_acc(zb_v, acc_s, s, width):
    def zrow(i, carry):
        for kk in range(width // 16):
            zb_v[i, pl.ds(16 * kk, 16)] = jnp.zeros((16,), _f32)
        return carry
    lax.fori_loop(0, ZR, zrow, 0)
    row0 = s * RPT
    for r in range(RPT // ZR):
        pltpu.sync_copy(zb_v, acc_s.at[pl.ds(row0 + r * ZR, ZR)])


def _edge_scores(xl_v, xr_v, att_v, exb_v, ns, shift_base, shift_off):
    lane = lax.iota(jnp.int32, 16)

    def group(t, carry):
        evec = jnp.zeros((16,), _f32)
        for j in range(16):
            i = t * 16 + j
            acc = None
            for kk in range(ns):
                sl = pl.ds(16 * kk, 16)
                u = xl_v[i, sl] + xr_v[i, sl]
                a = jnp.maximum(u, 0.2 * u) * att_v[sl]
                acc = a if acc is None else acc + a
            shv = jnp.where(lane == shift_off,
                            xr_v[i, pl.ds(shift_base, 16)], 0.0)
            e = jnp.sum(acc - shv)
            evec = jnp.where(lane == j, e, evec)
        exb_v[pl.ds(16 * t, 16)] = jnp.exp(evec)
        return carry
    lax.fori_loop(0, B // 16, group, 0)


@functools.partial(
    pl.kernel,
    out_type=jax.ShapeDtypeStruct((ETP,), _f32),
    mesh=_MESH,
    compiler_params=_SC_PARAMS,
    scratch_types=[
        pltpu.VMEM((B,), jnp.int32),
        pltpu.VMEM((B,), jnp.int32),
        pltpu.VMEM((B, 384), _f32),
        pltpu.VMEM((B, 384), _f32),
        pltpu.VMEM((304,), _f32),
        pltpu.VMEM((B,), _f32),
        pltpu.SemaphoreType.DMA,
    ],
)
def _l1_scores(src_hbm, dst_hbm, xl_hbm, xr_hbm, att_hbm, ex_hbm,
               idxs_v, idxd_v, xl_v, xr_v, att_v, exb_v, sem):
    c = lax.axis_index("c")
    s = lax.axis_index("s")
    pltpu.sync_copy(att_hbm.at[pl.ds(0, 304)], att_v)
    tbase = (c * 16 + s) * EPT

    def chunk(g, carry):
        base = tbase + g * B
        pltpu.sync_copy(src_hbm.at[pl.ds(base, B)], idxs_v)
        pltpu.sync_copy(dst_hbm.at[pl.ds(base, B)], idxd_v)
        cp1 = pltpu.async_copy(xl_hbm.at[idxs_v], xl_v, sem)
        cp2 = pltpu.async_copy(xr_hbm.at[idxd_v], xr_v, sem)
        cp1.wait()
        cp2.wait()
        _edge_scores(xl_v, xr_v, att_v, exb_v, 19, 288, 12)
        pltpu.sync_copy(exb_v, ex_hbm.at[pl.ds(base, B)])
        return carry
    lax.fori_loop(0, NCH, chunk, 0)


@functools.partial(
    pl.kernel,
    out_type=jax.ShapeDtypeStruct((2, NP, 128), _f32),
    mesh=_MESH,
    compiler_params=_SC_PARAMS,
    scratch_types=[
        pltpu.VMEM((B,), jnp.int32),
        pltpu.VMEM((B,), jnp.int32),
        pltpu.VMEM((B,), jnp.int32),
        pltpu.VMEM((B, 128), _f32),
        pltpu.VMEM((B,), _f32),
        pltpu.VMEM((B, 128), _f32),
        pltpu.VMEM((ZR, 128), _f32),
        pltpu.VMEM_SHARED((NP, 128), _f32),
        pltpu.SemaphoreType.DMA,
    ],
)
def _l1_agg_a(src_hbm, dst_hbm, ex_hbm, xlt_hbm, acc_hbm,
              idxs_v, ie_v, idxd_v, xl_v, exb_v, out_v, zb_v, acc_s, sem):
    c = lax.axis_index("c")
    s = lax.axis_index("s")
    _zero_acc(zb_v, acc_s, s, 128)
    plsc.subcore_barrier()
    tbase = s * EPS
    lane = lax.iota(jnp.int32, 16)

    def chunk(g, carry):
        base = tbase + g * B
        pltpu.sync_copy(src_hbm.at[pl.ds(base, B)], idxs_v)
        pltpu.sync_copy(dst_hbm.at[pl.ds(base, B)], idxd_v)
        pltpu.sync_copy(ex_hbm.at[pl.ds(base, B)], exb_v)
        for t in range(B // 16):
            sl = pl.ds(16 * t, 16)
            ie_v[sl] = idxs_v[sl] + c * NP
        pltpu.async_copy(xlt_hbm.at[ie_v], xl_v, sem).wait()

        def rgroup(t, carry2):
            exv = exb_v[pl.ds(16 * t, 16)]
            for j in range(16):
                i = t * 16 + j
                exs = jnp.sum(jnp.where(lane == j, exv, 0.0))
                for kk in range(8):
                    sl = pl.ds(16 * kk, 16)
                    out_v[i, sl] = exs * xl_v[i, sl]
            return carry2
        lax.fori_loop(0, B // 16, rgroup, 0)
        pltpu.sync_copy(out_v, acc_s.at[idxd_v], add=True)
        return carry
    lax.fori_loop(0, NCHS, chunk, 0)
    plsc.subcore_barrier()
    row0 = s * RPT
    pltpu.sync_copy(acc_s.at[pl.ds(row0, RPT)], acc_hbm.at[c, pl.ds(row0, RPT)])


@functools.partial(
    pl.kernel,
    out_type=jax.ShapeDtypeStruct((2, NP, 128), _f32),
    mesh=_MESH,
    compiler_params=_SC_PARAMS,
    scratch_types=[
        pltpu.VMEM((B,), jnp.int32),
        pltpu.VMEM((B,), jnp.int32),
        pltpu.VMEM((B,), jnp.int32),
        pltpu.VMEM((B, 128), _f32),
        pltpu.VMEM((B,), _f32),
        pltpu.VMEM((B, 128), _f32),
        pltpu.VMEM((ZR, 128), _f32),
        pltpu.VMEM_SHARED((NP, 128), _f32),
        pltpu.SemaphoreType.DMA,
    ],
)
def _l1_agg_b(src_hbm, dst_hbm, ex_hbm, xlt_hbm, acc_hbm,
              idxs_v, ie_v, idxd_v, xl_v, exb_v, out_v, zb_v, acc_s, sem):
    c = lax.axis_index("c")
    s = lax.axis_index("s")
    _zero_acc(zb_v, acc_s, s, 128)
    plsc.subcore_barrier()
    tbase = (c * 16 + s) * EPT
    lane = lax.iota(jnp.int32, 16)

    def chunk(g, carry):
        base = tbase + g * B
        pltpu.sync_copy(src_hbm.at[pl.ds(base, B)], idxs_v)
        pltpu.sync_copy(dst_hbm.at[pl.ds(base, B)], idxd_v)
        pltpu.sync_copy(ex_hbm.at[pl.ds(base, B)], exb_v)
        for t in range(B // 16):
            sl = pl.ds(16 * t, 16)
            ie_v[sl] = idxs_v[sl] + 2 * NP
        pltpu.async_copy(xlt_hbm.at[ie_v], xl_v, sem).wait()

        def rgroup(t, carry2):
            exv = exb_v[pl.ds(16 * t, 16)]
            for j in range(16):
                i = t * 16 + j
                exs = jnp.sum(jnp.where(lane == j, exv, 0.0))
                for kk in range(8):
                    sl = pl.ds(16 * kk, 16)
                    rowv = exs * xl_v[i, sl]
                    if kk == 2:
                        rowv = rowv + jnp.where(lane == 12, exs, 0.0)
                    out_v[i, sl] = rowv
            return carry2
        lax.fori_loop(0, B // 16, rgroup, 0)
        pltpu.sync_copy(out_v, acc_s.at[idxd_v], add=True)
        return carry
    lax.fori_loop(0, NCH, chunk, 0)
    plsc.subcore_barrier()
    row0 = s * RPT
    pltpu.sync_copy(acc_s.at[pl.ds(row0, RPT)], acc_hbm.at[c, pl.ds(row0, RPT)])


@functools.partial(
    pl.kernel,
    out_type=jax.ShapeDtypeStruct((2, NP, 128), _f32),
    mesh=_MESH,
    compiler_params=_SC_PARAMS,
    scratch_types=[
        pltpu.VMEM((B,), jnp.int32),
        pltpu.VMEM((B,), jnp.int32),
        pltpu.VMEM((B, 128), _f32),
        pltpu.VMEM((B, 128), _f32),
        pltpu.VMEM((128,), _f32),
        pltpu.VMEM((B,), _f32),
        pltpu.VMEM((B, 128), _f32),
        pltpu.VMEM((ZR, 128), _f32),
        pltpu.VMEM_SHARED((NP, 128), _f32),
        pltpu.SemaphoreType.DMA,
    ],
)
def _l2_fused(src_hbm, dst_hbm, xl_hbm, xr_hbm, att_hbm, acc_hbm,
              idxs_v, idxd_v, xl_v, xr_v, att_v, exb_v, out_v, zb_v,
              acc_s, sem):
    c = lax.axis_index("c")
    s = lax.axis_index("s")
    _zero_acc(zb_v, acc_s, s, 128)
    pltpu.sync_copy(att_hbm.at[pl.ds(0, 128)], att_v)
    plsc.subcore_barrier()
    tbase = (c * 16 + s) * EPT
    lane = lax.iota(jnp.int32, 16)

    def chunk(g, carry):
        base = tbase + g * B
        pltpu.sync_copy(src_hbm.at[pl.ds(base, B)], idxs_v)
        pltpu.sync_copy(dst_hbm.at[pl.ds(base, B)], idxd_v)
        cp1 = pltpu.async_copy(xl_hbm.at[idxs_v], xl_v, sem)
        cp2 = pltpu.async_copy(xr_hbm.at[idxd_v], xr_v, sem)
        cp1.wait()
        cp2.wait()
        _edge_scores(xl_v, xr_v, att_v, exb_v, 7, 96, 4)

        def rgroup(t, carry2):
            exv = exb_v[pl.ds(16 * t, 16)]
            for j in range(16):
                i = t * 16 + j
                exs = jnp.sum(jnp.where(lane == j, exv, 0.0))
                for kk in range(7):
                    sl = pl.ds(16 * kk, 16)
                    out_v[i, sl] = exs * xl_v[i, sl]
                out_v[i, pl.ds(112, 16)] = jnp.where(lane == 0, exs, 0.0)
            return carry2
        lax.fori_loop(0, B // 16, rgroup, 0)
        pltpu.sync_copy(out_v, acc_s.at[idxd_v], add=True)
        return carry
    lax.fori_loop(0, NCH, chunk, 0)
    plsc.subcore_barrier()
    row0 = s * RPT
    pltpu.sync_copy(acc_s.at[pl.ds(row0, RPT)], acc_hbm.at[c, pl.ds(row0, RPT)])


@functools.partial(
    pl.kernel,
    out_type=jax.ShapeDtypeStruct((2, 2, NP), _f32),
    mesh=_MESH,
    compiler_params=_SC_PARAMS,
    scratch_types=[
        pltpu.VMEM((B,), jnp.int32),
        pltpu.VMEM((B,), jnp.int32),
        pltpu.VMEM((NP,), _f32),
        pltpu.VMEM((NP,), _f32),
        pltpu.VMEM((NP,), _f32),
        pltpu.VMEM((16,), _f32),
        pltpu.VMEM((NP,), _f32),
        pltpu.VMEM((NP,), _f32),
        pltpu.VMEM((RPT,), _f32),
        pltpu.VMEM((RPT,), _f32),
        pltpu.VMEM_SHARED((16, NP), _f32),
        pltpu.VMEM_SHARED((16, NP), _f32),
        pltpu.SemaphoreType.DMA,
    ],
)
def _l3_fused(src_hbm, dst_hbm, xl_hbm, xr_hbm, esh_hbm, att_hbm, acc_hbm,
              idxs_v, idxd_v, xl_v, xr_v, esh_v, att_v, num_v, den_v,
              tmp_v, res_v, shn_s, shd_s, sem):
    c = lax.axis_index("c")
    s = lax.axis_index("s")
    pltpu.sync_copy(xl_hbm, xl_v)
    pltpu.sync_copy(xr_hbm, xr_v)
    pltpu.sync_copy(esh_hbm, esh_v)
    pltpu.sync_copy(att_hbm, att_v)

    def zrow(i, carry):
        num_v[pl.ds(16 * i, 16)] = jnp.zeros((16,), _f32)
        den_v[pl.ds(16 * i, 16)] = jnp.zeros((16,), _f32)
        return carry
    lax.fori_loop(0, NP // 16, zrow, 0)

    tbase = (c * 16 + s) * EPT
    attv = att_v[pl.ds(0, 16)]

    def chunk(g, carry):
        base = tbase + g * B
        pltpu.sync_copy(src_hbm.at[pl.ds(base, B)], idxs_v)
        pltpu.sync_copy(dst_hbm.at[pl.ds(base, B)], idxd_v)
        for t in range(B // 16):
            sl = pl.ds(16 * t, 16)
            ids = idxs_v[sl]
            idd = idxd_v[sl]
            lv = plsc.load_gather(xl_v, [ids])
            rv = plsc.load_gather(xr_v, [idd])
            ev = plsc.load_gather(esh_v, [idd])
            u = lv + rv
            ex = jnp.exp(jnp.maximum(u, 0.2 * u) * attv - ev)
            plsc.addupdate_scatter(num_v, [idd], ex * lv)
            plsc.addupdate_scatter(den_v, [idd], ex)
        return carry
    lax.fori_loop(0, NCH, chunk, 0)

    pltpu.sync_copy(num_v, shn_s.at[s])
    pltpu.sync_copy(den_v, shd_s.at[s])
    plsc.subcore_barrier()
    col0 = s * RPT
    for which in range(2):
        sh = shn_s if which == 0 else shd_s
        dst = res_v

        def addrow(j, carry):
            pltpu.sync_copy(sh.at[j, pl.ds(col0, RPT)], tmp_v)
            for kk in range(RPT // 16):
                sl = pl.ds(16 * kk, 16)
                prev = jnp.where(j == 0, jnp.zeros((16,), _f32), dst[sl])
                dst[sl] = prev + tmp_v[sl]
            return carry
        lax.fori_loop(0, 16, addrow, 0)
        pltpu.sync_copy(dst, acc_hbm.at[c, which, pl.ds(col0, RPT)])


@functools.partial(
    pl.kernel,
    out_type=jax.ShapeDtypeStruct((ELP,), _f32),
    mesh=_MESH,
    compiler_params=_SC_PARAMS,
    scratch_types=[
        pltpu.VMEM((B,), jnp.int32),
        pltpu.VMEM((B,), jnp.int32),
        pltpu.VMEM((B, 128), _f32),
        pltpu.VMEM((B, 128), _f32),
        pltpu.VMEM((B,), _f32),
        pltpu.SemaphoreType.DMA,
    ],
)
def _pair_dots(ia_hbm, ib_hbm, z_hbm, d_hbm,
               idxa_v, idxb_v, za_v, zb_v, db_v, sem):
    c = lax.axis_index("c")
    s = lax.axis_index("s")
    tbase = (c * 16 + s) * LPT
    lane = lax.iota(jnp.int32, 16)

    def chunk(g, carry):
        base = tbase + g * B
        pltpu.sync_copy(ia_hbm.at[pl.ds(base, B)], idxa_v)
        pltpu.sync_copy(ib_hbm.at[pl.ds(base, B)], idxb_v)
        cp1 = pltpu.async_copy(z_hbm.at[idxa_v], za_v, sem)
        cp2 = pltpu.async_copy(z_hbm.at[idxb_v], zb_v, sem)
        cp1.wait()
        cp2.wait()

        def egroup(t, carry2):
            dvec = jnp.zeros((16,), _f32)
            for j in range(16):
                i = t * 16 + j
                acc = None
                for kk in range(7):
                    sl = pl.ds(16 * kk, 16)
                    a = za_v[i, sl] * zb_v[i, sl]
                    acc = a if acc is None else acc + a
                dvec = jnp.where(lane == j, jnp.sum(acc), dvec)
            db_v[pl.ds(16 * t, 16)] = dvec
            return carry2
        lax.fori_loop(0, B // 16, egroup, 0)
        pltpu.sync_copy(db_v, d_hbm.at[pl.ds(base, B)])
        return carry
    lax.fori_loop(0, LCH, chunk, 0)


def _dot(a, b):
    return jnp.dot(a, b, preferred_element_type=_f32)


def _tc1_body(x_ref, wl_ref, wr_ref, att_ref, xlf_ref, xre_ref, xlt_ref):
    xb = x_ref[:]
    xl = _dot(xb, wl_ref[:])
    xr = _dot(xb, wr_ref[:])
    u = xl + xr
    esh = jnp.sum(jnp.maximum(u, 0.2 * u) * att_ref[:], axis=1, keepdims=True)
    lane = lax.broadcasted_iota(jnp.int32, xr.shape, 1)
    xre_ref[:] = jnp.where(lane == 300, esh, xr)
    xlf_ref[:] = xl
    xlt_ref[0] = xl[:, 0:128]
    xlt_ref[1] = xl[:, 128:256]
    xlt_ref[2] = xl[:, 256:384]


def _tc2_body(acca_ref, accb_ref, wl_ref, wr_ref, att_ref, b1_ref,
              xl2_ref, xr2_ref):
    bsum = accb_ref[0] + accb_ref[1]
    sden = bsum[:, 44:45]
    hin = jnp.concatenate(
        [acca_ref[0], acca_ref[1], bsum[:, :64]], axis=1)
    h = jnp.maximum(jnp.where(sden > 0, hin / sden, 0.0) + b1_ref[:], 0.0)
    xl2 = _dot(h, wl_ref[:])
    xr2 = _dot(h, wr_ref[:])
    u = xl2 + xr2
    esh = jnp.sum(jnp.maximum(u, 0.2 * u) * att_ref[:], axis=1, keepdims=True)
    lane = lax.broadcasted_iota(jnp.int32, xr2.shape, 1)
    xl2_ref[:] = xl2
    xr2_ref[:] = jnp.where(lane == 100, esh, xr2)


def _tc3_body(acc_ref, x_ref, wlin1_ref, blin1_ref, wlin2_ref, blin2_ref,
              b2_ref, wl3_ref, wr3_ref, att3_ref, z_ref, xl3_ref, xr3_ref,
              esh3_ref):
    a = acc_ref[0] + acc_ref[1]
    sden = a[:, 112:113]
    x1 = jnp.maximum(jnp.where(sden > 0, a / sden, 0.0) + b2_ref[:], 0.0)
    xb = x_ref[:]
    xs = x1 + jnp.maximum(_dot(xb, wlin1_ref[:]) + blin1_ref[:], 0.0)
    z = x1 + jnp.maximum(_dot(xb, wlin2_ref[:]) + blin2_ref[:], 0.0)
    lane = lax.broadcasted_iota(jnp.int32, z.shape, 1)
    z_ref[:] = jnp.where(lane < 100, z, 0.0)
    xl3 = _dot(xs, wl3_ref[:])
    xr3 = _dot(xs, wr3_ref[:])
    u = xl3 + xr3
    esh3 = jnp.maximum(u, 0.2 * u) * att3_ref[:]
    xl3_ref[:] = xl3
    xr3_ref[:] = xr3
    esh3_ref[:] = esh3


def _tc4_body(acc_ref, d_ref, b3_ref, out_ref, loss_ref):
    a = acc_ref[:]
    num = a[0:1, :] + a[2:3, :]
    den = a[1:2, :] + a[3:4, :]
    out_ref[:] = jnp.where(den > 0, num / den, 0.0) + b3_ref[0, 0]
    dmat = d_ref[:]
    sig = jax.nn.sigmoid(dmat)
    tpos = jnp.log(sig + 1e-15)
    tneg = jnp.log(1.0 - sig)
    rid = lax.broadcasted_iota(jnp.int32, dmat.shape, 0)
    psum = jnp.sum(jnp.where(rid < 1250, tpos, 0.0))
    nsum = jnp.sum(jnp.where((rid >= 1250) & (rid < 2500), tneg, 0.0))
    loss_ref[:] = jnp.reshape(-(psum / E) - (nsum / E), (1, 1))


def _row_spec(rb, w):
    return pl.BlockSpec((rb, w), lambda i: (i, 0))


def _full_spec(shape):
    nd = len(shape)
    return pl.BlockSpec(shape, lambda i: (0,) * nd)


def kernel(x, edge_index, neg_edge_index, Wl1, Wr1, att1, b1, Wl2, Wr2, att2,
           b2, Wl3, Wr3, att3, b3, Wlin1, blin1, Wlin2, blin2, c1, c2):
    i32 = jnp.int32
    xp = jnp.pad(x, ((0, NP - N), (0, 0)))
    loop = jnp.arange(N, dtype=edge_index.dtype)
    srcp = jnp.concatenate(
        [edge_index[0], loop, jnp.zeros((ETP - ETOT,), i32)])
    dstp = jnp.concatenate(
        [edge_index[1], loop, jnp.full((ETP - ETOT,), N, i32)])
    ia = jnp.concatenate(
        [edge_index[0], neg_edge_index[0], jnp.zeros((ELP - 2 * E,), i32)])
    ib = jnp.concatenate(
        [edge_index[1], neg_edge_index[1], jnp.zeros((ELP - 2 * E,), i32)])

    wl1p = jnp.pad(Wl1, ((0, 0), (0, 84)))
    wr1p = jnp.pad(Wr1, ((0, 0), (0, 84)))
    att1p = jnp.pad(att1, (0, 84))
    b1p = jnp.pad(b1, (0, 20)).reshape(1, 320)
    wl2p = jnp.pad(Wl2, ((0, 20), (0, 28)))
    wr2p = jnp.pad(Wr2, ((0, 20), (0, 28)))
    att2p = jnp.pad(att2, (0, 28))
    b2p = jnp.pad(b2, (0, 28)).reshape(1, 128)
    wl3p = jnp.pad(Wl3, ((0, 28), (0, 0)))
    wr3p = jnp.pad(Wr3, ((0, 28), (0, 0)))
    att3b = jnp.full((16,), att3[0], _f32)
    b3r = b3.reshape(1, 1)
    wlin1p = jnp.pad(Wlin1, ((0, 0), (0, 28)))
    blin1p = jnp.pad(blin1, (0, 28)).reshape(1, 128)
    wlin2p = jnp.pad(Wlin2, ((0, 0), (0, 28)))
    blin2p = jnp.pad(blin2, (0, 28)).reshape(1, 128)

    rb = RPT

    xl1f, xr1e, xlt = pl.pallas_call(
        _tc1_body,
        grid=(16,),
        in_specs=[_row_spec(rb, 128), _full_spec((128, 384)),
                  _full_spec((128, 384)), _full_spec((1, 384))],
        out_specs=[_row_spec(rb, 384), _row_spec(rb, 384),
                   pl.BlockSpec((3, rb, 128), lambda i: (0, i, 0))],
        out_shape=[jax.ShapeDtypeStruct((NP, 384), _f32),
                   jax.ShapeDtypeStruct((NP, 384), _f32),
                   jax.ShapeDtypeStruct((3, NP, 128), _f32)],
    )(xp, wl1p, wr1p, att1p.reshape(1, 384))

    xlt = xlt.reshape(3 * NP, 128)
    exbuf = _l1_scores(srcp, dstp, xl1f, xr1e, att1p)
    acc1a = _l1_agg_a(srcp, dstp, exbuf, xlt)
    acc1b = _l1_agg_b(srcp, dstp, exbuf, xlt)

    xl2p, xr2e = pl.pallas_call(
        _tc2_body,
        grid=(16,),
        in_specs=[pl.BlockSpec((2, rb, 128), lambda i: (0, i, 0)),
                  pl.BlockSpec((2, rb, 128), lambda i: (0, i, 0)),
                  _full_spec((320, 128)), _full_spec((320, 128)),
                  _full_spec((1, 128)), _full_spec((1, 320))],
        out_specs=[_row_spec(rb, 128), _row_spec(rb, 128)],
        out_shape=[jax.ShapeDtypeStruct((NP, 128), _f32),
                   jax.ShapeDtypeStruct((NP, 128), _f32)],
    )(acc1a, acc1b, wl2p, wr2p, att2p.reshape(1, 128), b1p)

    acc2 = _l2_fused(srcp, dstp, xl2p, xr2e, att2p)

    zp, xl3c, xr3c, esh3c = pl.pallas_call(
        _tc3_body,
        grid=(16,),
        in_specs=[pl.BlockSpec((2, rb, 128), lambda i: (0, i, 0)),
                  _row_spec(rb, 128),
                  _full_spec((128, 128)), _full_spec((1, 128)),
                  _full_spec((128, 128)), _full_spec((1, 128)),
                  _full_spec((1, 128)),
                  _full_spec((128, 1)), _full_spec((128, 1)),
                  _full_spec((1, 1))],
        out_specs=[_row_spec(rb, 128), _row_spec(rb, 1), _row_spec(rb, 1),
                   _row_spec(rb, 1)],
        out_shape=[jax.ShapeDtypeStruct((NP, 128), _f32),
                   jax.ShapeDtypeStruct((NP, 1), _f32),
                   jax.ShapeDtypeStruct((NP, 1), _f32),
                   jax.ShapeDtypeStruct((NP, 1), _f32)],
    )(acc2, xp, wlin1p, blin1p, wlin2p, blin2p, b2p, wl3p, wr3p,
      att3[0].reshape(1, 1))

    acc3 = _l3_fused(srcp, dstp, xl3c.reshape(NP), xr3c.reshape(NP),
                     esh3c.reshape(NP), att3b)
    dots = _pair_dots(ia, ib, zp)

    out_np, loss = pl.pallas_call(
        _tc4_body,
        grid=(1,),
        in_specs=[_full_spec((4, NP)), _full_spec((ELP // 128, 128)),
                  _full_spec((1, 1))],
        out_specs=[_full_spec((1, NP)), _full_spec((1, 1))],
        out_shape=[jax.ShapeDtypeStruct((1, NP), _f32),
                   jax.ShapeDtypeStruct((1, 1), _f32)],
    )(acc3.reshape(4, NP), dots.reshape(ELP // 128, 128), b3r)

    return (out_np.reshape(NP, 1)[:N], loss[0, 0], c1, c2)


# speedup vs baseline: 7.2976x; 1.3157x over previous
import functools

import jax
import jax.numpy as jnp
from jax import lax
from jax.experimental import pallas as pl
from jax.experimental.pallas import tpu as pltpu
from jax.experimental.pallas import tpu_sc as plsc

N = 10000
D = 128
E = 160000
NP = 10240
RPT = NP // 16
B = 64
ETOT = E + N
ETP = 172032
EPT = ETP // 32
NCH = EPT // B
EPS = ETP // 16
NCHS = EPS // B
NROW = ETP // B
ELP = 323584
LPT = ELP // 32
LCH = LPT // B
LROW = ELP // B

_MESH = plsc.VectorSubcoreMesh(core_axis_name="c", subcore_axis_name="s")
_SC_PARAMS = pltpu.CompilerParams(needs_layout_passes=False)
_f32 = jnp.float32


def _zero_acc(zb_v, acc_s, s, width):
    def zrow(i, carry):
        for kk in range(width // 16):
            zb_v[i, pl.ds(16 * kk, 16)] = jnp.zeros((16,), _f32)
        return carry
    lax.fori_loop(0, B, zrow, 0)
    row0 = s * RPT
    for r in range(RPT // B):
        pltpu.sync_copy(zb_v, acc_s.at[pl.ds(row0 + r * B, B)])


def _edge_scores(xl_v, xr_v, att_v, out_ref, obase, ns, shift_base,
                 shift_off):
    lane = lax.iota(jnp.int32, 16)
    for t in range(B // 16):
        evec = jnp.zeros((16,), _f32)
        for j in range(16):
            i = t * 16 + j
            acc = None
            for kk in range(ns):
                sl = pl.ds(16 * kk, 16)
                u = xl_v[i, sl] + xr_v[i, sl]
                a = jnp.maximum(u, 0.2 * u) * att_v[sl]
                acc = a if acc is None else acc + a
            shv = jnp.where(lane == shift_off,
                            xr_v[i, pl.ds(shift_base, 16)], 0.0)
            e = jnp.sum(acc - shv)
            evec = jnp.where(lane == j, e, evec)
        out_ref[pl.ds(obase + 16 * t, 16)] = jnp.exp(evec)


def _copy_row(src_v, dst_v, row):
    for t in range(B // 16):
        sl = pl.ds(16 * t, 16)
        dst_v[sl] = src_v[row, sl]


def _drain(src, dst, sem):
    pltpu.make_async_copy(src, dst, sem).wait()


@functools.partial(
    pl.kernel,
    out_type=jax.ShapeDtypeStruct((ETP,), _f32),
    mesh=_MESH,
    compiler_params=_SC_PARAMS,
    scratch_types=[
        pltpu.VMEM((2, B), jnp.int32),
        pltpu.VMEM((2, B), jnp.int32),
        pltpu.VMEM((B, 384), _f32),
        pltpu.VMEM((B, 384), _f32),
        pltpu.VMEM((B, 384), _f32),
        pltpu.VMEM((B, 384), _f32),
        pltpu.VMEM((304,), _f32),
        pltpu.VMEM((EPT,), _f32),
        pltpu.SemaphoreType.DMA,
        pltpu.SemaphoreType.DMA,
        pltpu.SemaphoreType.DMA,
        pltpu.SemaphoreType.DMA,
    ],
)
def _l1_scores(ep_hbm, xl_hbm, xr_hbm, att_hbm, ex_hbm,
               ep0, ep1, xl0, xr0, xl1, xr1, att_v, exa_v,
               seme0, seme1, semg0, semg1):
    c = lax.axis_index("c")
    s = lax.axis_index("s")
    pltpu.sync_copy(att_hbm.at[pl.ds(0, 304)], att_v)
    tb = (c * 16 + s) * NCH
    last = NROW - 1

    pltpu.sync_copy(ep_hbm.at[tb], ep0)
    c1 = pltpu.async_copy(xl_hbm.at[ep0.at[0]], xl0, semg0)
    c2 = pltpu.async_copy(xr_hbm.at[ep0.at[1]], xr0, semg0)

    def body(m, carry):
        g1 = tb + 2 * m + 1
        g2 = jnp.minimum(tb + 2 * m + 2, last)
        pltpu.async_copy(ep_hbm.at[g1], ep1, seme1)
        _drain(xl_hbm.at[ep0.at[0]], xl0, semg0)
        _drain(xr_hbm.at[ep0.at[1]], xr0, semg0)
        _drain(ep_hbm.at[0], ep1, seme1)
        pltpu.async_copy(xl_hbm.at[ep1.at[0]], xl1, semg1)
        pltpu.async_copy(xr_hbm.at[ep1.at[1]], xr1, semg1)
        _edge_scores(xl0, xr0, att_v, exa_v, (2 * m) * B, 19, 288, 12)
        pltpu.async_copy(ep_hbm.at[g2], ep0, seme0)
        _drain(xl_hbm.at[ep1.at[0]], xl1, semg1)
        _drain(xr_hbm.at[ep1.at[1]], xr1, semg1)
        _drain(ep_hbm.at[0], ep0, seme0)
        pltpu.async_copy(xl_hbm.at[ep0.at[0]], xl0, semg0)
        pltpu.async_copy(xr_hbm.at[ep0.at[1]], xr0, semg0)
        _edge_scores(xl1, xr1, att_v, exa_v, (2 * m + 1) * B, 19, 288, 12)
        return carry
    lax.fori_loop(0, NCH // 2, body, 0)
    _drain(xl_hbm.at[ep0.at[0]], xl0, semg0)
    _drain(xr_hbm.at[ep0.at[1]], xr0, semg0)
    pltpu.sync_copy(exa_v, ex_hbm.at[pl.ds(tb * B, EPT)])


def _make_l1_agg(core_offset, edge_split):
    nch = NCH if edge_split else NCHS

    @functools.partial(
        pl.kernel,
        out_type=jax.ShapeDtypeStruct((2, NP, 128), _f32),
        mesh=_MESH,
        compiler_params=_SC_PARAMS,
        scratch_types=[
            pltpu.VMEM((2, B), jnp.int32),
            pltpu.VMEM((2, B), jnp.int32),
            pltpu.VMEM((B,), jnp.int32),
            pltpu.VMEM((B,), jnp.int32),
            pltpu.VMEM((B,), jnp.int32),
            pltpu.VMEM((B, 128), _f32),
            pltpu.VMEM((B, 128), _f32),
            pltpu.VMEM((B,), _f32),
            pltpu.VMEM((B,), _f32),
            pltpu.VMEM((B,), _f32),
            pltpu.VMEM((B, 128), _f32),
            pltpu.VMEM_SHARED((NP, 128), _f32),
            pltpu.SemaphoreType.DMA,
            pltpu.SemaphoreType.DMA,
            pltpu.SemaphoreType.DMA,
            pltpu.SemaphoreType.DMA,
            pltpu.SemaphoreType.DMA,
        ],
    )
    def k(ep_hbm, ex_hbm, xlt_hbm, acc_hbm,
          ep0, ep1, ie0, ie1, idd, xl0, xl1, exb0, exb1, exq, out_v, acc_s,
          seme0, seme1, semg0, semg1, semsc):
        c = lax.axis_index("c")
        s = lax.axis_index("s")
        _zero_acc(out_v, acc_s, s, 128)
        plsc.subcore_barrier()
        off = core_offset(c) * NP
        tb = ((c * 16 + s) * NCH) if edge_split else (s * NCHS)
        last = NROW - 1
        lane = lax.iota(jnp.int32, 16)

        def fire(epb, ieb, exb, xlb, row, seme, semg, sync):
            if sync:
                pltpu.sync_copy(ep_hbm.at[row], epb)
                pltpu.sync_copy(ex_hbm.at[pl.ds(row * B, B)], exb)
            for t in range(B // 16):
                sl = pl.ds(16 * t, 16)
                ieb[sl] = epb[0, sl] + off
            pltpu.async_copy(xlt_hbm.at[ieb], xlb, semg)

        def rows(epb, xlb, exb, grow, denom_here):
            for t in range(B // 16):
                exv = exb[pl.ds(16 * t, 16)]
                for j in range(16):
                    i = t * 16 + j
                    exs = jnp.sum(jnp.where(lane == j, exv, 0.0))
                    for kk in range(8):
                        sl = pl.ds(16 * kk, 16)
                        rowv = exs * xlb[i, sl]
                        if denom_here and kk == 2:
                            rowv = rowv + jnp.where(lane == 12, exs, 0.0)
                        out_v[i, sl] = rowv
            _copy_row(epb, idd, 1)
            pltpu.sync_copy(out_v, acc_s.at[idd], add=True)

        fire(ep0, ie0, exb0, xl0, tb, seme0, semg0, True)

        def body(m, carry):
            g1 = tb + 2 * m + 1
            g2 = jnp.minimum(tb + 2 * m + 2, last)
            pltpu.async_copy(ep_hbm.at[g1], ep1, seme1)
            pltpu.async_copy(ex_hbm.at[pl.ds(g1 * B, B)], exb1, seme1)
            _drain(xlt_hbm.at[ie0], xl0, semg0)
            _drain(ep_hbm.at[0], ep1, seme1)
            _drain(ex_hbm.at[pl.ds(0, B)], exb1, seme1)
            fire(ep1, ie1, exb1, xl1, g1, seme1, semg1, False)
            rows(ep0, xl0, exb0, tb + 2 * m, edge_split)
            pltpu.async_copy(ep_hbm.at[g2], ep0, seme0)
            pltpu.async_copy(ex_hbm.at[pl.ds(g2 * B, B)], exb0, seme0)
            _drain(xlt_hbm.at[ie1], xl1, semg1)
            _drain(ep_hbm.at[0], ep0, seme0)
            _drain(ex_hbm.at[pl.ds(0, B)], exb0, seme0)
            fire(ep0, ie0, exb0, xl0, g2, seme0, semg0, False)
            rows(ep1, xl1, exb1, tb + 2 * m + 1, edge_split)
            return carry
        lax.fori_loop(0, nch // 2, body, 0)
        _drain(xlt_hbm.at[ie0], xl0, semg0)
        plsc.subcore_barrier()
        row0 = s * RPT
        pltpu.sync_copy(acc_s.at[pl.ds(row0, RPT)],
                        acc_hbm.at[c, pl.ds(row0, RPT)])
    return k


_l1_agg_a = _make_l1_agg(lambda c: c, edge_split=False)
_l1_agg_b = _make_l1_agg(lambda c: c * 0 + 2, edge_split=True)


@functools.partial(
    pl.kernel,
    out_type=jax.ShapeDtypeStruct((2, NP, 128), _f32),
    mesh=_MESH,
    compiler_params=_SC_PARAMS,
    scratch_types=[
        pltpu.VMEM((2, B), jnp.int32),
        pltpu.VMEM((2, B), jnp.int32),
        pltpu.VMEM((B,), jnp.int32),
        pltpu.VMEM((B, 128), _f32),
        pltpu.VMEM((B, 128), _f32),
        pltpu.VMEM((B, 128), _f32),
        pltpu.VMEM((B, 128), _f32),
        pltpu.VMEM((128,), _f32),
        pltpu.VMEM((B,), _f32),
        pltpu.VMEM((B, 128), _f32),
        pltpu.VMEM_SHARED((NP, 128), _f32),
        pltpu.SemaphoreType.DMA,
        pltpu.SemaphoreType.DMA,
        pltpu.SemaphoreType.DMA,
        pltpu.SemaphoreType.DMA,
    ],
)
def _l2_fused(ep_hbm, xl_hbm, xr_hbm, att_hbm, acc_hbm,
              ep0, ep1, idd, xl0, xr0, xl1, xr1, att_v, exb_v, out_v, acc_s,
              seme0, seme1, semg0, semg1):
    c = lax.axis_index("c")
    s = lax.axis_index("s")
    _zero_acc(out_v, acc_s, s, 128)
    pltpu.sync_copy(att_hbm.at[pl.ds(0, 128)], att_v)
    plsc.subcore_barrier()
    tb = (c * 16 + s) * NCH
    last = NROW - 1
    lane = lax.iota(jnp.int32, 16)

    def rows(epb, xlb, m):
        for t in range(B // 16):
            exv = exb_v[pl.ds(16 * t, 16)]
            for j in range(16):
                i = t * 16 + j
                exs = jnp.sum(jnp.where(lane == j, exv, 0.0))
                for kk in range(7):
                    sl = pl.ds(16 * kk, 16)
                    out_v[i, sl] = exs * xlb[i, sl]
                out_v[i, pl.ds(112, 16)] = jnp.where(lane == 0, exs, 0.0)
        _copy_row(epb, idd, 1)
        pltpu.sync_copy(out_v, acc_s.at[idd], add=True)

    pltpu.sync_copy(ep_hbm.at[tb], ep0)
    pltpu.async_copy(xl_hbm.at[ep0.at[0]], xl0, semg0)
    pltpu.async_copy(xr_hbm.at[ep0.at[1]], xr0, semg0)

    def body(m, carry):
        g1 = tb + 2 * m + 1
        g2 = jnp.minimum(tb + 2 * m + 2, last)
        pltpu.async_copy(ep_hbm.at[g1], ep1, seme1)
        _drain(xl_hbm.at[ep0.at[0]], xl0, semg0)
        _drain(xr_hbm.at[ep0.at[1]], xr0, semg0)
        _drain(ep_hbm.at[0], ep1, seme1)
        pltpu.async_copy(xl_hbm.at[ep1.at[0]], xl1, semg1)
        pltpu.async_copy(xr_hbm.at[ep1.at[1]], xr1, semg1)
        _edge_scores(xl0, xr0, att_v, exb_v, 0, 7, 96, 4)
        rows(ep0, xl0, 2 * m)
        pltpu.async_copy(ep_hbm.at[g2], ep0, seme0)
        _drain(xl_hbm.at[ep1.at[0]], xl1, semg1)
        _drain(xr_hbm.at[ep1.at[1]], xr1, semg1)
        _drain(ep_hbm.at[0], ep0, seme0)
        pltpu.async_copy(xl_hbm.at[ep0.at[0]], xl0, semg0)
        pltpu.async_copy(xr_hbm.at[ep0.at[1]], xr0, semg0)
        _edge_scores(xl1, xr1, att_v, exb_v, 0, 7, 96, 4)
        rows(ep1, xl1, 2 * m + 1)
        return carry
    lax.fori_loop(0, NCH // 2, body, 0)
    _drain(xl_hbm.at[ep0.at[0]], xl0, semg0)
    _drain(xr_hbm.at[ep0.at[1]], xr0, semg0)
    plsc.subcore_barrier()
    row0 = s * RPT
    pltpu.sync_copy(acc_s.at[pl.ds(row0, RPT)], acc_hbm.at[c, pl.ds(row0, RPT)])


@functools.partial(
    pl.kernel,
    out_type=jax.ShapeDtypeStruct((2, 2, NP), _f32),
    mesh=_MESH,
    compiler_params=_SC_PARAMS,
    scratch_types=[
        pltpu.VMEM((2, B), jnp.int32),
        pltpu.VMEM((2, B), jnp.int32),
        pltpu.VMEM((NP,), _f32),
        pltpu.VMEM((NP,), _f32),
        pltpu.VMEM((NP,), _f32),
        pltpu.VMEM((16,), _f32),
        pltpu.VMEM((NP,), _f32),
        pltpu.VMEM((NP,), _f32),
        pltpu.VMEM((RPT,), _f32),
        pltpu.VMEM((RPT,), _f32),
        pltpu.VMEM_SHARED((16, NP), _f32),
        pltpu.VMEM_SHARED((16, NP), _f32),
        pltpu.SemaphoreType.DMA,
        pltpu.SemaphoreType.DMA,
    ],
)
def _l3_fused(ep_hbm, xl_hbm, xr_hbm, esh_hbm, att_hbm, acc_hbm,
              ep0, ep1, xl_v, xr_v, esh_v, att_v, num_v, den_v,
              tmp_v, res_v, shn_s, shd_s, seme0, seme1):
    c = lax.axis_index("c")
    s = lax.axis_index("s")
    pltpu.sync_copy(xl_hbm, xl_v)
    pltpu.sync_copy(xr_hbm, xr_v)
    pltpu.sync_copy(esh_hbm, esh_v)
    pltpu.sync_copy(att_hbm, att_v)

    def zrow(i, carry):
        num_v[pl.ds(16 * i, 16)] = jnp.zeros((16,), _f32)
        den_v[pl.ds(16 * i, 16)] = jnp.zeros((16,), _f32)
        return carry
    lax.fori_loop(0, NP // 16, zrow, 0)

    tb = (c * 16 + s) * NCH
    last = NROW - 1
    attv = att_v[pl.ds(0, 16)]

    def compute(epb):
        for t in range(B // 16):
            sl = pl.ds(16 * t, 16)
            ids = epb[0, sl]
            idd = epb[1, sl]
            lv = plsc.load_gather(xl_v, [ids])
            rv = plsc.load_gather(xr_v, [idd])
            ev = plsc.load_gather(esh_v, [idd])
            u = lv + rv
            ex = jnp.exp(jnp.maximum(u, 0.2 * u) * attv - ev)
            plsc.addupdate_scatter(num_v, [idd], ex * lv)
            plsc.addupdate_scatter(den_v, [idd], ex)

    pltpu.sync_copy(ep_hbm.at[tb], ep0)

    def body(m, carry):
        g1 = tb + 2 * m + 1
        g2 = jnp.minimum(tb + 2 * m + 2, last)
        pltpu.async_copy(ep_hbm.at[g1], ep1, seme1)
        compute(ep0)
        _drain(ep_hbm.at[0], ep1, seme1)
        pltpu.async_copy(ep_hbm.at[g2], ep0, seme0)
        compute(ep1)
        _drain(ep_hbm.at[0], ep0, seme0)
        return carry
    lax.fori_loop(0, NCH // 2, body, 0)

    pltpu.sync_copy(num_v, shn_s.at[s])
    pltpu.sync_copy(den_v, shd_s.at[s])
    plsc.subcore_barrier()
    col0 = s * RPT
    for which in range(2):
        sh = shn_s if which == 0 else shd_s
        dst = res_v

        def addrow(j, carry):
            pltpu.sync_copy(sh.at[j, pl.ds(col0, RPT)], tmp_v)
            for kk in range(RPT // 16):
                sl = pl.ds(16 * kk, 16)
                prev = jnp.where(j == 0, jnp.zeros((16,), _f32), dst[sl])
                dst[sl] = prev + tmp_v[sl]
            return carry
        lax.fori_loop(0, 16, addrow, 0)
        pltpu.sync_copy(dst, acc_hbm.at[c, which, pl.ds(col0, RPT)])


@functools.partial(
    pl.kernel,
    out_type=jax.ShapeDtypeStruct((ELP,), _f32),
    mesh=_MESH,
    compiler_params=_SC_PARAMS,
    scratch_types=[
        pltpu.VMEM((2, B), jnp.int32),
        pltpu.VMEM((2, B), jnp.int32),
        pltpu.VMEM((B, 128), _f32),
        pltpu.VMEM((B, 128), _f32),
        pltpu.VMEM((B, 128), _f32),
        pltpu.VMEM((B, 128), _f32),
        pltpu.VMEM((LPT,), _f32),
        pltpu.SemaphoreType.DMA,
        pltpu.SemaphoreType.DMA,
        pltpu.SemaphoreType.DMA,
        pltpu.SemaphoreType.DMA,
    ],
)
def _pair_dots(ep_hbm, z_hbm, d_hbm,
               ep0, ep1, za0, zb0, za1, zb1, da_v,
               seme0, seme1, semg0, semg1):
    c = lax.axis_index("c")
    s = lax.axis_index("s")
    tb = (c * 16 + s) * LCH
    last = LROW - 1
    lane = lax.iota(jnp.int32, 16)

    def compute(za, zb, m):
        for t in range(B // 16):
            dvec = jnp.zeros((16,), _f32)
            for j in range(16):
                i = t * 16 + j
                acc = None
                for kk in range(7):
                    sl = pl.ds(16 * kk, 16)
                    a = za[i, sl] * zb[i, sl]
                    acc = a if acc is None else acc + a
                dvec = jnp.where(lane == j, jnp.sum(acc), dvec)
            da_v[pl.ds(m * B + 16 * t, 16)] = dvec

    pltpu.sync_copy(ep_hbm.at[tb], ep0)
    pltpu.async_copy(z_hbm.at[ep0.at[0]], za0, semg0)
    pltpu.async_copy(z_hbm.at[ep0.at[1]], zb0, semg0)

    def body(m, carry):
        g1 = tb + 2 * m + 1
        g2 = jnp.minimum(tb + 2 * m + 2, last)
        pltpu.async_copy(ep_hbm.at[g1], ep1, seme1)
        _drain(z_hbm.at[ep0.at[0]], za0, semg0)
        _drain(z_hbm.at[ep0.at[1]], zb0, semg0)
        _drain(ep_hbm.at[0], ep1, seme1)
        pltpu.async_copy(z_hbm.at[ep1.at[0]], za1, semg1)
        pltpu.async_copy(z_hbm.at[ep1.at[1]], zb1, semg1)
        compute(za0, zb0, 2 * m)
        pltpu.async_copy(ep_hbm.at[g2], ep0, seme0)
        _drain(z_hbm.at[ep1.at[0]], za1, semg1)
        _drain(z_hbm.at[ep1.at[1]], zb1, semg1)
        _drain(ep_hbm.at[0], ep0, seme0)
        pltpu.async_copy(z_hbm.at[ep0.at[0]], za0, semg0)
        pltpu.async_copy(z_hbm.at[ep0.at[1]], zb0, semg0)
        compute(za1, zb1, 2 * m + 1)
        return carry
    lax.fori_loop(0, LCH // 2, body, 0)
    _drain(z_hbm.at[ep0.at[0]], za0, semg0)
    _drain(z_hbm.at[ep0.at[1]], zb0, semg0)
    pltpu.sync_copy(da_v, d_hbm.at[pl.ds(tb * B, LPT)])


def _dot(a, b):
    return jnp.dot(a, b, preferred_element_type=_f32)


def _tc1_body(x_ref, wl_ref, wr_ref, att_ref, xlf_ref, xre_ref, xlt_ref):
    xb = x_ref[:]
    xl = _dot(xb, wl_ref[:])
    xr = _dot(xb, wr_ref[:])
    u = xl + xr
    esh = jnp.sum(jnp.maximum(u, 0.2 * u) * att_ref[:], axis=1, keepdims=True)
    lane = lax.broadcasted_iota(jnp.int32, xr.shape, 1)
    xre_ref[:] = jnp.where(lane == 300, esh, xr)
    xlf_ref[:] = xl
    xlt_ref[0] = xl[:, 0:128]
    xlt_ref[1] = xl[:, 128:256]
    xlt_ref[2] = xl[:, 256:384]


def _tc2_body(acca_ref, accb_ref, wl_ref, wr_ref, att_ref, b1_ref,
              xl2_ref, xr2_ref):
    bsum = accb_ref[0] + accb_ref[1]
    sden = bsum[:, 44:45]
    hin = jnp.concatenate(
        [acca_ref[0], acca_ref[1], bsum[:, :64]], axis=1)
    h = jnp.maximum(jnp.where(sden > 0, hin / sden, 0.0) + b1_ref[:], 0.0)
    xl2 = _dot(h, wl_ref[:])
    xr2 = _dot(h, wr_ref[:])
    u = xl2 + xr2
    esh = jnp.sum(jnp.maximum(u, 0.2 * u) * att_ref[:], axis=1, keepdims=True)
    lane = lax.broadcasted_iota(jnp.int32, xr2.shape, 1)
    xl2_ref[:] = xl2
    xr2_ref[:] = jnp.where(lane == 100, esh, xr2)


def _tc3_body(acc_ref, x_ref, wlin1_ref, blin1_ref, wlin2_ref, blin2_ref,
              b2_ref, wl3_ref, wr3_ref, att3_ref, z_ref, xl3_ref, xr3_ref,
              esh3_ref):
    a = acc_ref[0] + acc_ref[1]
    sden = a[:, 112:113]
    x1 = jnp.maximum(jnp.where(sden > 0, a / sden, 0.0) + b2_ref[:], 0.0)
    xb = x_ref[:]
    xs = x1 + jnp.maximum(_dot(xb, wlin1_ref[:]) + blin1_ref[:], 0.0)
    z = x1 + jnp.maximum(_dot(xb, wlin2_ref[:]) + blin2_ref[:], 0.0)
    lane = lax.broadcasted_iota(jnp.int32, z.shape, 1)
    z_ref[:] = jnp.where(lane < 100, z, 0.0)
    xl3 = _dot(xs, wl3_ref[:])
    xr3 = _dot(xs, wr3_ref[:])
    u = xl3 + xr3
    esh3 = jnp.maximum(u, 0.2 * u) * att3_ref[:]
    xl3_ref[:] = xl3
    xr3_ref[:] = xr3
    esh3_ref[:] = esh3


def _tc4_body(acc_ref, d_ref, b3_ref, out_ref, loss_ref):
    a = acc_ref[:]
    num = a[0:1, :] + a[2:3, :]
    den = a[1:2, :] + a[3:4, :]
    out_ref[:] = jnp.where(den > 0, num / den, 0.0) + b3_ref[0, 0]
    dmat = d_ref[:]
    sig = jax.nn.sigmoid(dmat)
    tpos = jnp.log(sig + 1e-15)
    tneg = jnp.log(1.0 - sig)
    rid = lax.broadcasted_iota(jnp.int32, dmat.shape, 0)
    psum = jnp.sum(jnp.where(rid < 1250, tpos, 0.0))
    nsum = jnp.sum(jnp.where((rid >= 1250) & (rid < 2500), tneg, 0.0))
    loss_ref[:] = jnp.reshape(-(psum / E) - (nsum / E), (1, 1))


def _row_spec(rb, w):
    return pl.BlockSpec((rb, w), lambda i: (i, 0))


def _full_spec(shape):
    nd = len(shape)
    return pl.BlockSpec(shape, lambda i: (0,) * nd)


def kernel(x, edge_index, neg_edge_index, Wl1, Wr1, att1, b1, Wl2, Wr2, att2,
           b2, Wl3, Wr3, att3, b3, Wlin1, blin1, Wlin2, blin2, c1, c2):
    i32 = jnp.int32
    xp = jnp.pad(x, ((0, NP - N), (0, 0)))
    loop = jnp.arange(N, dtype=edge_index.dtype)
    srcp = jnp.concatenate(
        [edge_index[0], loop, jnp.zeros((ETP - ETOT,), i32)])
    dstp = jnp.concatenate(
        [edge_index[1], loop, jnp.full((ETP - ETOT,), N, i32)])
    epk = jnp.stack([srcp.reshape(NROW, B), dstp.reshape(NROW, B)], axis=1)
    ia = jnp.concatenate(
        [edge_index[0], neg_edge_index[0], jnp.zeros((ELP - 2 * E,), i32)])
    ib = jnp.concatenate(
        [edge_index[1], neg_edge_index[1], jnp.zeros((ELP - 2 * E,), i32)])
    lpk = jnp.stack([ia.reshape(LROW, B), ib.reshape(LROW, B)], axis=1)

    wl1p = jnp.pad(Wl1, ((0, 0), (0, 84)))
    wr1p = jnp.pad(Wr1, ((0, 0), (0, 84)))
    att1p = jnp.pad(att1, (0, 84))
    b1p = jnp.pad(b1, (0, 20)).reshape(1, 320)
    wl2p = jnp.pad(Wl2, ((0, 20), (0, 28)))
    wr2p = jnp.pad(Wr2, ((0, 20), (0, 28)))
    att2p = jnp.pad(att2, (0, 28))
    b2p = jnp.pad(b2, (0, 28)).reshape(1, 128)
    wl3p = jnp.pad(Wl3, ((0, 28), (0, 0)))
    wr3p = jnp.pad(Wr3, ((0, 28), (0, 0)))
    att3b = jnp.full((16,), att3[0], _f32)
    b3r = b3.reshape(1, 1)
    wlin1p = jnp.pad(Wlin1, ((0, 0), (0, 28)))
    blin1p = jnp.pad(blin1, (0, 28)).reshape(1, 128)
    wlin2p = jnp.pad(Wlin2, ((0, 0), (0, 28)))
    blin2p = jnp.pad(blin2, (0, 28)).reshape(1, 128)

    rb = RPT

    xl1f, xr1e, xlt = pl.pallas_call(
        _tc1_body,
        grid=(16,),
        in_specs=[_row_spec(rb, 128), _full_spec((128, 384)),
                  _full_spec((128, 384)), _full_spec((1, 384))],
        out_specs=[_row_spec(rb, 384), _row_spec(rb, 384),
                   pl.BlockSpec((3, rb, 128), lambda i: (0, i, 0))],
        out_shape=[jax.ShapeDtypeStruct((NP, 384), _f32),
                   jax.ShapeDtypeStruct((NP, 384), _f32),
                   jax.ShapeDtypeStruct((3, NP, 128), _f32)],
    )(xp, wl1p, wr1p, att1p.reshape(1, 384))

    xlt = xlt.reshape(3 * NP, 128)
    exbuf = _l1_scores(epk, xl1f, xr1e, att1p)
    acc1a = _l1_agg_a(epk, exbuf, xlt)
    acc1b = _l1_agg_b(epk, exbuf, xlt)

    xl2p, xr2e = pl.pallas_call(
        _tc2_body,
        grid=(16,),
        in_specs=[pl.BlockSpec((2, rb, 128), lambda i: (0, i, 0)),
                  pl.BlockSpec((2, rb, 128), lambda i: (0, i, 0)),
                  _full_spec((320, 128)), _full_spec((320, 128)),
                  _full_spec((1, 128)), _full_spec((1, 320))],
        out_specs=[_row_spec(rb, 128), _row_spec(rb, 128)],
        out_shape=[jax.ShapeDtypeStruct((NP, 128), _f32),
                   jax.ShapeDtypeStruct((NP, 128), _f32)],
    )(acc1a, acc1b, wl2p, wr2p, att2p.reshape(1, 128), b1p)

    acc2 = _l2_fused(epk, xl2p, xr2e, att2p)

    zp, xl3c, xr3c, esh3c = pl.pallas_call(
        _tc3_body,
        grid=(16,),
        in_specs=[pl.BlockSpec((2, rb, 128), lambda i: (0, i, 0)),
                  _row_spec(rb, 128),
                  _full_spec((128, 128)), _full_spec((1, 128)),
                  _full_spec((128, 128)), _full_spec((1, 128)),
                  _full_spec((1, 128)),
                  _full_spec((128, 1)), _full_spec((128, 1)),
                  _full_spec((1, 1))],
        out_specs=[_row_spec(rb, 128), _row_spec(rb, 1), _row_spec(rb, 1),
                   _row_spec(rb, 1)],
        out_shape=[jax.ShapeDtypeStruct((NP, 128), _f32),
                   jax.ShapeDtypeStruct((NP, 1), _f32),
                   jax.ShapeDtypeStruct((NP, 1), _f32),
                   jax.ShapeDtypeStruct((NP, 1), _f32)],
    )(acc2, xp, wlin1p, blin1p, wlin2p, blin2p, b2p, wl3p, wr3p,
      att3[0].reshape(1, 1))

    acc3 = _l3_fused(epk, xl3c.reshape(NP), xr3c.reshape(NP),
                     esh3c.reshape(NP), att3b)
    dots = _pair_dots(lpk, zp)

    out_np, loss = pl.pallas_call(
        _tc4_body,
        grid=(1,),
        in_specs=[_full_spec((4, NP)), _full_spec((ELP // 128, 128)),
                  _full_spec((1, 1))],
        out_specs=[_full_spec((1, NP)), _full_spec((1, 1))],
        out_shape=[jax.ShapeDtypeStruct((1, NP), _f32),
                   jax.ShapeDtypeStruct((1, 1), _f32)],
    )(acc3.reshape(4, NP), dots.reshape(ELP // 128, 128), b3r)

    return (out_np.reshape(NP, 1)[:N], loss[0, 0], c1, c2)
